# Initial kernel scaffold; baseline (speedup 1.0000x reference)
import jax
import jax.numpy as jnp
from jax.experimental import pallas as pl


def kernel(new_xyz, pointset, feature):
    raise NotImplementedError("write your pallas kernel here")


# speedup vs baseline: 13.8612x; 13.8612x over previous
import functools

import jax
import jax.numpy as jnp
import numpy as np
from jax import lax
from jax.experimental import pallas as pl
from jax.experimental.pallas import tpu as pltpu
from jax.experimental.pallas import tpu_sc as plsc

_RADIUS2 = np.float32(0.2 * 0.2)
_NS = 32
_N = 4096
_M = 1024
_B = 8
_C = 64
_OUTW = 3 + _C
_NSUB = 32
_QPW = (_B * _M) // _NSUB
_GQ = 4
_GS = _GQ * _NS


def _body(qx_h, qy_h, qz_h, px_h, py_h, pz_h, tab_h, out_h,
          qxv, qyv, qzv, pxv, pyv, pzv, idxq, idxf, rows, obuf, sem):
    wid = lax.axis_index("c") * 16 + lax.axis_index("s")
    qbase = wid * _QPW
    b = wid // (_M // _QPW)
    pbase = b * _N

    pltpu.sync_copy(qx_h.at[pl.ds(qbase, _QPW)], qxv)
    pltpu.sync_copy(qy_h.at[pl.ds(qbase, _QPW)], qyv)
    pltpu.sync_copy(qz_h.at[pl.ds(qbase, _QPW)], qzv)
    pltpu.sync_copy(px_h.at[pl.ds(pbase, _N)], pxv)
    pltpu.sync_copy(py_h.at[pl.ds(pbase, _N)], pyv)
    pltpu.sync_copy(pz_h.at[pl.ds(pbase, _N)], pzv)

    iota = jnp.arange(16, dtype=jnp.int32)
    zero16 = jnp.zeros((16,), jnp.int32)
    one16 = jnp.ones((16,), jnp.int32)
    boffv = jnp.full((16,), pbase, jnp.int32)

    def per_query(j, _):
        jsp = jnp.full((16,), j, jnp.int32)
        qxb = plsc.load_gather(qxv, [jsp])
        qyb = plsc.load_gather(qyv, [jsp])
        qzb = plsc.load_gather(qzv, [jsp])

        def cond(carry):
            c, cnt = carry
            return jnp.logical_and(cnt < _NS, c < _N // 16)

        def step(carry):
            c, cnt = carry
            base = c * 16
            dx = pxv[pl.ds(base, 16)] - qxb
            dy = pyv[pl.ds(base, 16)] - qyb
            dz = pzv[pl.ds(base, 16)] - qzb
            d2 = dx * dx + dy * dy + dz * dz
            m = d2 < _RADIUS2
            mi = jnp.where(m, one16, zero16)
            ranks = plsc.cumsum(mi)
            dest = ranks + (cnt - 1)
            plsc.store_scatter(idxq, [dest], iota + base, mask=m)
            return c + 1, cnt + jnp.sum(mi)

        _, cnt = lax.while_loop(cond, step, (jnp.int32(0), jnp.int32(0)))

        cntv = jnp.full((16,), cnt, jnp.int32)
        f16 = idxq[pl.ds(0, 16)]
        first = plsc.cummax(jnp.where(iota == 0, f16, -one16))
        first = jnp.where(cntv > 0, first, jnp.full((16,), _N - 1, jnp.int32))
        for t in range(_NS // 16):
            slots = iota + (16 * t)
            cur = idxq[pl.ds(16 * t, 16)]
            val = jnp.where(slots < cntv, cur, first) + boffv
            idxf[pl.ds(j * _NS + 16 * t, 16)] = val
        return _

    lax.fori_loop(0, _QPW, per_query, 0)

    cols = [iota + 16 * cc for cc in range(_C // 16)]

    def per_group(g, _):
        pltpu.async_copy(tab_h.at[idxf.at[pl.ds(g * _GS, _GS)]], rows,
                         sem).wait()

        def per_row(r, _):
            rsp = jnp.full((16,), r, jnp.int32)
            for cc in range(_C // 16):
                v = plsc.load_gather(rows, [rsp, cols[cc]])
                plsc.store_scatter(obuf, [rsp, cols[cc] + 3], v)
            return _

        lax.fori_loop(0, _GS, per_row, 0)

        for t in range(_GS // 16):
            jq = g * _GQ + t // 2
            jsp = jnp.full((16,), jq, jnp.int32)
            lidx = idxf[pl.ds(g * _GS + t * 16, 16)] - boffv
            lx = plsc.load_gather(pxv, [lidx]) - plsc.load_gather(qxv, [jsp])
            ly = plsc.load_gather(pyv, [lidx]) - plsc.load_gather(qyv, [jsp])
            lz = plsc.load_gather(pzv, [lidx]) - plsc.load_gather(qzv, [jsp])
            rowv = iota + t * 16
            plsc.store_scatter(obuf, [rowv, zero16], lx)
            plsc.store_scatter(obuf, [rowv, one16], ly)
            plsc.store_scatter(obuf, [rowv, one16 + one16], lz)

        q0 = qbase + g * _GQ
        pltpu.sync_copy(obuf, out_h.at[pl.ds(q0 * _NS, _GS)])
        return _

    lax.fori_loop(0, _QPW // _GQ, per_group, 0)


@jax.jit
def _run(qx, qy, qz, px, py, pz, table):
    mesh = plsc.VectorSubcoreMesh(core_axis_name="c", subcore_axis_name="s")
    f = functools.partial(
        pl.kernel,
        out_type=jax.ShapeDtypeStruct((_B * _M * _NS, _OUTW), jnp.float32),
        mesh=mesh,
        compiler_params=pltpu.CompilerParams(needs_layout_passes=False,
                                             use_tc_tiling_on_sc=False),
        scratch_types=[
            pltpu.VMEM((_QPW,), jnp.float32),
            pltpu.VMEM((_QPW,), jnp.float32),
            pltpu.VMEM((_QPW,), jnp.float32),
            pltpu.VMEM((_N,), jnp.float32),
            pltpu.VMEM((_N,), jnp.float32),
            pltpu.VMEM((_N,), jnp.float32),
            pltpu.VMEM((_NS + 16,), jnp.int32),
            pltpu.VMEM((_QPW * _NS,), jnp.int32),
            pltpu.VMEM((_GS, _C), jnp.float32),
            pltpu.VMEM((_GS, _OUTW), jnp.float32),
            pltpu.SemaphoreType.DMA,
        ],
    )(_body)
    return f(qx, qy, qz, px, py, pz, table)


def kernel(new_xyz, pointset, feature):
    qx = new_xyz[..., 0].reshape(-1)
    qy = new_xyz[..., 1].reshape(-1)
    qz = new_xyz[..., 2].reshape(-1)
    px = pointset[..., 0].reshape(-1)
    py = pointset[..., 1].reshape(-1)
    pz = pointset[..., 2].reshape(-1)
    table = feature.reshape(-1, _C)
    out = _run(qx, qy, qz, px, py, pz, table)
    return out.reshape(_B, _M, _NS, _OUTW)


# speedup vs baseline: 17.6871x; 1.2760x over previous
import functools

import jax
import jax.numpy as jnp
import numpy as np
from jax import lax
from jax.experimental import pallas as pl
from jax.experimental.pallas import tpu as pltpu
from jax.experimental.pallas import tpu_sc as plsc

_RADIUS2 = np.float32(0.2 * 0.2)
_NS = 32
_N = 4096
_M = 1024
_B = 8
_C = 64
_OUTW = 3 + _C
_NSUB = 32
_QPW = (_B * _M) // _NSUB
_GQ = 4
_GS = _GQ * _NS
_SUP = 8


def _body(qx_h, qy_h, qz_h, px_h, py_h, pz_h, tab_h, out_h,
          qxv, qyv, qzv, pxv, pyv, pzv, idxq, idxf, rows, obuf, sem):
    wid = lax.axis_index("c") * 16 + lax.axis_index("s")
    qbase = wid * _QPW
    b = wid // (_M // _QPW)
    pbase = b * _N

    pltpu.sync_copy(qx_h.at[pl.ds(qbase, _QPW)], qxv)
    pltpu.sync_copy(qy_h.at[pl.ds(qbase, _QPW)], qyv)
    pltpu.sync_copy(qz_h.at[pl.ds(qbase, _QPW)], qzv)
    pltpu.sync_copy(px_h.at[pl.ds(pbase, _N)], pxv)
    pltpu.sync_copy(py_h.at[pl.ds(pbase, _N)], pyv)
    pltpu.sync_copy(pz_h.at[pl.ds(pbase, _N)], pzv)

    iota = jnp.arange(16, dtype=jnp.int32)
    zero16 = jnp.zeros((16,), jnp.int32)
    one16 = jnp.ones((16,), jnp.int32)
    boffv = jnp.full((16,), pbase, jnp.int32)

    def per_query(j, _):
        jsp = jnp.full((16,), j, jnp.int32)
        qxb = plsc.load_gather(qxv, [jsp])
        qyb = plsc.load_gather(qyv, [jsp])
        qzb = plsc.load_gather(qzv, [jsp])

        def cond(carry):
            s, _cv, cnt_s = carry
            return jnp.logical_and(cnt_s < _NS, s < _N // (16 * _SUP))

        def step(carry):
            s, cnt_v, _cs = carry
            for u in range(_SUP):
                base = s * (16 * _SUP) + u * 16
                dx = pxv[pl.ds(base, 16)] - qxb
                dy = pyv[pl.ds(base, 16)] - qyb
                dz = pzv[pl.ds(base, 16)] - qzb
                d2 = dx * dx + dy * dy + dz * dz
                m = d2 < _RADIUS2
                mi = jnp.where(m, one16, zero16)
                dest = plsc.cumsum(mi) + (cnt_v - one16)
                plsc.store_scatter(idxq, [dest], iota + base, mask=m)
                cnt_v = cnt_v + plsc.all_reduce_population_count(m)
            return s + 1, cnt_v, jnp.max(cnt_v)

        _s_fin, _cv_fin, cnt = lax.while_loop(
            cond, step, (jnp.int32(0), zero16, jnp.int32(0)))

        cntv = jnp.full((16,), cnt, jnp.int32)
        f16 = idxq[pl.ds(0, 16)]
        first = plsc.cummax(jnp.where(iota == 0, f16, -one16))
        first = jnp.where(cntv > 0, first, jnp.full((16,), _N - 1, jnp.int32))
        for t in range(_NS // 16):
            slots = iota + (16 * t)
            cur = idxq[pl.ds(16 * t, 16)]
            val = jnp.where(slots < cntv, cur, first) + boffv
            idxf[pl.ds(j * _NS + 16 * t, 16)] = val
        return _

    lax.fori_loop(0, _QPW, per_query, 0)

    def per_group(g, _):
        pltpu.async_copy(tab_h.at[idxf.at[pl.ds(g * _GS, _GS)]], rows,
                         sem).wait()

        @plsc.parallel_loop(0, _GS, unroll=4)
        def per_row(r):
            base = r * _OUTW + 3
            for cc in range(_C // 16):
                v = rows[r, pl.ds(16 * cc, 16)]
                plsc.store_scatter(obuf, [iota + (base + 16 * cc)], v)

        for t in range(_GS // 16):
            jq = g * _GQ + t // 2
            jsp = jnp.full((16,), jq, jnp.int32)
            lidx = idxf[pl.ds(g * _GS + t * 16, 16)] - boffv
            lx = plsc.load_gather(pxv, [lidx]) - plsc.load_gather(qxv, [jsp])
            ly = plsc.load_gather(pyv, [lidx]) - plsc.load_gather(qyv, [jsp])
            lz = plsc.load_gather(pzv, [lidx]) - plsc.load_gather(qzv, [jsp])
            rdest = (iota + t * 16) * _OUTW
            plsc.store_scatter(obuf, [rdest], lx)
            plsc.store_scatter(obuf, [rdest + 1], ly)
            plsc.store_scatter(obuf, [rdest + 2], lz)

        q0 = qbase + g * _GQ
        pltpu.sync_copy(obuf, out_h.at[pl.ds(q0 * _NS * _OUTW, _GS * _OUTW)])
        return _

    lax.fori_loop(0, _QPW // _GQ, per_group, 0)


@jax.jit
def _run(qx, qy, qz, px, py, pz, table):
    mesh = plsc.VectorSubcoreMesh(core_axis_name="c", subcore_axis_name="s")
    f = functools.partial(
        pl.kernel,
        out_type=jax.ShapeDtypeStruct((_B * _M * _NS * _OUTW,), jnp.float32),
        mesh=mesh,
        compiler_params=pltpu.CompilerParams(needs_layout_passes=False,
                                             use_tc_tiling_on_sc=False),
        scratch_types=[
            pltpu.VMEM((_QPW,), jnp.float32),
            pltpu.VMEM((_QPW,), jnp.float32),
            pltpu.VMEM((_QPW,), jnp.float32),
            pltpu.VMEM((_N,), jnp.float32),
            pltpu.VMEM((_N,), jnp.float32),
            pltpu.VMEM((_N,), jnp.float32),
            pltpu.VMEM((_NS + 16 * _SUP,), jnp.int32),
            pltpu.VMEM((_QPW * _NS,), jnp.int32),
            pltpu.VMEM((_GS, _C), jnp.float32),
            pltpu.VMEM((_GS * _OUTW,), jnp.float32),
            pltpu.SemaphoreType.DMA,
        ],
    )(_body)
    return f(qx, qy, qz, px, py, pz, table)


def kernel(new_xyz, pointset, feature):
    qx = new_xyz[..., 0].reshape(-1)
    qy = new_xyz[..., 1].reshape(-1)
    qz = new_xyz[..., 2].reshape(-1)
    px = pointset[..., 0].reshape(-1)
    py = pointset[..., 1].reshape(-1)
    pz = pointset[..., 2].reshape(-1)
    table = feature.reshape(-1, _C)
    out = _run(qx, qy, qz, px, py, pz, table)
    return out.reshape(_B, _M, _NS, _OUTW)
